# baseline (speedup 1.0000x reference)
import functools

import jax
import jax.numpy as jnp
from jax import lax
from jax.experimental import pallas as pl
from jax.experimental.pallas import tpu as pltpu
from jax.experimental.pallas import tpu_sc as plsc

B = 8
V = 100000
F = 200000

VP = 102400
NC, NS = 2, 16
NW = NC * NS
FPT = 6400
FP = FPT * NW
CH = 640
NCHUNK = FPT // CH
GA = 128
NGA = CH // GA


def _products_body(x_ref, y_ref, z_ref, o_ref):
    x = x_ref[...]
    y = y_ref[...]
    z = z_ref[...]
    o_ref[0, :, :] = x * y
    o_ref[1, :, :] = x * z
    o_ref[2, :, :] = z * y


def _products(x, y, z):
    blk = 2048
    grid = VP // blk
    return pl.pallas_call(
        _products_body,
        grid=(grid,),
        in_specs=[pl.BlockSpec((B, blk), lambda i: (0, i))] * 3,
        out_specs=pl.BlockSpec((3, B, blk), lambda i: (0, 0, i)),
        out_shape=jax.ShapeDtypeStruct((3, B, VP), jnp.float32),
    )(x, y, z)


def _gather_sum(table, faces1):
    mesh = plsc.VectorSubcoreMesh(core_axis_name="c", subcore_axis_name="s")

    @functools.partial(
        pl.kernel,
        mesh=mesh,
        compiler_params=pltpu.CompilerParams(use_tc_tiling_on_sc=False),
        out_type=jax.ShapeDtypeStruct((3 * B, F, 1), jnp.float32),
        scratch_types=[
            pltpu.VMEM((3 * CH,), jnp.int32),
            pltpu.VMEM((3 * CH,), jnp.int32),
            pltpu.VMEM((3 * CH, 24), jnp.float32),
            pltpu.VMEM((3 * CH, 24), jnp.float32),
            pltpu.VMEM((CH, 24), jnp.float32),
            pltpu.SemaphoreType.DMA,
            pltpu.SemaphoreType.DMA,
            pltpu.SemaphoreType.DMA,
        ],
    )
    def k(table_hbm, faces_hbm, out_hbm, idx0, idx1, rr0, rr1, o_s, sg0, sg1, ss):
        wid = lax.axis_index("s") * NC + lax.axis_index("c")
        fbase0 = wid * FPT
        idx = (idx0, idx1)
        rr = (rr0, rr1)
        sg = (sg0, sg1)
        ii = lax.iota(jnp.int32, 16)
        v3 = ii * 3

        def chunk_fb(kk):
            fb_n = fbase0 + kk * CH
            return fb_n < F, pl.multiple_of(jnp.minimum(fb_n, F - CH), 8)

        def fire(kk, p):
            alive, fb = chunk_fb(kk)

            @pl.when(alive)
            def _():
                pltpu.sync_copy(faces_hbm.at[pl.ds(3 * fb, 3 * CH)], idx[p])
                for j in range(3 * CH // GA):
                    pltpu.async_copy(
                        table_hbm.at[idx[p].at[pl.ds(j * GA, GA)]],
                        rr[p].at[pl.ds(j * GA, GA), :],
                        sg[p],
                    )

        def wait_gathers(kk, p):
            alive, _ = chunk_fb(kk)

            @pl.when(alive)
            def _():
                for j in range(3 * CH // GA):
                    pltpu.make_async_copy(
                        table_hbm.at[idx[p].at[pl.ds(j * GA, GA)]],
                        rr[p].at[pl.ds(j * GA, GA), :],
                        sg[p],
                    ).wait()

        def compute(kk, p):
            alive, _ = chunk_fb(kk)
            rp = rr[p]

            @pl.when(alive)
            def _():
                def body(i, carry):
                    for h in (0, 8):
                        sl = pl.ds(h, 16)
                        o_s[i, sl] = 1.0 - (
                            rp[3 * i, sl] + rp[3 * i + 1, sl] + rp[3 * i + 2, sl]
                        )
                    return carry

                lax.fori_loop(0, CH, body, 0)

        def fire_stores(kk):
            alive, fb = chunk_fb(kk)

            @pl.when(alive)
            def _():
                for c in range(3):
                    for b in range(B):
                        pltpu.async_copy(
                            o_s.at[:, pl.ds(c * B + b, 1)],
                            out_hbm.at[c * B + b, pl.ds(fb, CH), :],
                            ss,
                        )

        def wait_stores(kk):
            alive, fb = chunk_fb(kk)

            @pl.when(alive)
            def _():
                for c in range(3):
                    for b in range(B):
                        pltpu.make_async_copy(
                            o_s.at[:, pl.ds(c * B + b, 1)],
                            out_hbm.at[c * B + b, pl.ds(fb, CH), :],
                            ss,
                        ).wait()

        fire(0, 0)
        for kk in range(NCHUNK):
            p = kk % 2
            if kk + 1 < NCHUNK:
                fire(kk + 1, 1 - p)
            wait_gathers(kk, p)
            if kk > 0:
                wait_stores(kk - 1)
            compute(kk, p)
            fire_stores(kk)
        wait_stores(NCHUNK - 1)

    return k(table, faces1)


def kernel(vertex_normals, faces):
    faces = jnp.squeeze(faces)
    x = jnp.pad(vertex_normals[:, :, 0], ((0, 0), (0, VP - V)))
    y = jnp.pad(vertex_normals[:, :, 1], ((0, 0), (0, VP - V)))
    z = jnp.pad(vertex_normals[:, :, 2], ((0, 0), (0, VP - V)))
    prods = _products(x, y, z)
    table = prods.transpose(2, 0, 1).reshape(VP, 3 * B)
    faces1 = faces.reshape(-1)
    return _gather_sum(table, faces1).reshape(3, B, F)


# speedup vs baseline: 25.2522x; 25.2522x over previous
import functools

import jax
import jax.numpy as jnp
from jax import lax
from jax.experimental import pallas as pl
from jax.experimental.pallas import tpu as pltpu
from jax.experimental.pallas import tpu_sc as plsc

B = 8
V = 100000
F = 200000

VP = 102400
NC, NS = 2, 16
NW = NC * NS
FPT = 6400
CH = 640
NCHUNK = FPT // CH
GA = 128


def _products_body(x_ref, y_ref, z_ref, o_ref):
    x = x_ref[...]
    y = y_ref[...]
    z = z_ref[...]
    o_ref[0, :, :] = x * y
    o_ref[1, :, :] = x * z
    o_ref[2, :, :] = z * y


def _products(x, y, z):
    blk = 2048
    return pl.pallas_call(
        _products_body,
        grid=(VP // blk,),
        in_specs=[pl.BlockSpec((B, blk), lambda i: (0, i))] * 3,
        out_specs=pl.BlockSpec((3, B, blk), lambda i: (0, 0, i)),
        out_shape=jax.ShapeDtypeStruct((3, B, VP), jnp.float32),
    )(x, y, z)


def _gather_sum(table, faces1):
    mesh = plsc.VectorSubcoreMesh(core_axis_name="c", subcore_axis_name="s")

    @functools.partial(
        pl.kernel,
        mesh=mesh,
        compiler_params=pltpu.CompilerParams(use_tc_tiling_on_sc=False),
        out_type=jax.ShapeDtypeStruct((F, 24), jnp.float32),
        scratch_types=[
            pltpu.VMEM((3 * CH,), jnp.int32),
            pltpu.VMEM((3 * CH,), jnp.int32),
            pltpu.VMEM((3 * CH, 24), jnp.float32),
            pltpu.VMEM((3 * CH, 24), jnp.float32),
            pltpu.VMEM((CH, 24), jnp.float32),
            pltpu.SemaphoreType.DMA,
            pltpu.SemaphoreType.DMA,
            pltpu.SemaphoreType.DMA,
        ],
    )
    def k(table_hbm, faces_hbm, out_hbm, idx0, idx1, rr0, rr1, o_s, sg0, sg1, ss):
        wid = lax.axis_index("s") * NC + lax.axis_index("c")
        fbase0 = wid * FPT
        idx = (idx0, idx1)
        rr = (rr0, rr1)
        sg = (sg0, sg1)

        def chunk_fb(kk):
            fb_n = fbase0 + kk * CH
            return fb_n < F, pl.multiple_of(jnp.minimum(fb_n, F - CH), 8)

        def fire(kk, p):
            alive, fb = chunk_fb(kk)

            @pl.when(alive)
            def _():
                pltpu.sync_copy(faces_hbm.at[pl.ds(3 * fb, 3 * CH)], idx[p])
                for j in range(3 * CH // GA):
                    pltpu.async_copy(
                        table_hbm.at[idx[p].at[pl.ds(j * GA, GA)]],
                        rr[p].at[pl.ds(j * GA, GA), :],
                        sg[p],
                    )

        def wait_gathers(kk, p):
            alive, _ = chunk_fb(kk)

            @pl.when(alive)
            def _():
                for j in range(3 * CH // GA):
                    pltpu.make_async_copy(
                        table_hbm.at[idx[p].at[pl.ds(j * GA, GA)]],
                        rr[p].at[pl.ds(j * GA, GA), :],
                        sg[p],
                    ).wait()

        def compute(kk, p):
            alive, _ = chunk_fb(kk)
            rp = rr[p]

            @pl.when(alive)
            def _():
                def body(i, carry):
                    for h in (0, 8):
                        sl = pl.ds(h, 16)
                        o_s[i, sl] = 1.0 - (
                            rp[3 * i, sl] + rp[3 * i + 1, sl] + rp[3 * i + 2, sl]
                        )
                    return carry

                lax.fori_loop(0, CH, body, 0)

        def fire_stores(kk):
            alive, fb = chunk_fb(kk)

            @pl.when(alive)
            def _():
                pltpu.async_copy(o_s, out_hbm.at[pl.ds(fb, CH), :], ss)

        def wait_stores(kk):
            alive, fb = chunk_fb(kk)

            @pl.when(alive)
            def _():
                pltpu.make_async_copy(
                    o_s, out_hbm.at[pl.ds(fb, CH), :], ss
                ).wait()

        fire(0, 0)
        for kk in range(NCHUNK):
            p = kk % 2
            if kk + 1 < NCHUNK:
                fire(kk + 1, 1 - p)
            wait_gathers(kk, p)
            if kk > 0:
                wait_stores(kk - 1)
            compute(kk, p)
            fire_stores(kk)
        wait_stores(NCHUNK - 1)

    return k(table, faces1)


def kernel(vertex_normals, faces):
    faces = jnp.squeeze(faces)
    x = jnp.pad(vertex_normals[:, :, 0], ((0, 0), (0, VP - V)))
    y = jnp.pad(vertex_normals[:, :, 1], ((0, 0), (0, VP - V)))
    z = jnp.pad(vertex_normals[:, :, 2], ((0, 0), (0, VP - V)))
    prods = _products(x, y, z)
    table = prods.transpose(2, 0, 1).reshape(VP, 3 * B)
    faces1 = faces.reshape(-1)
    out24 = _gather_sum(table, faces1)
    return out24.T.reshape(3, B, F)
